# Initial kernel scaffold; baseline (speedup 1.0000x reference)
import functools

import jax
import jax.numpy as jnp
from jax import lax
from jax.experimental import pallas as pl
from jax.experimental.pallas import tpu as pltpu
from jax.experimental.pallas import tpu_sc as plsc

N = 10000
E = 320000
D = 128

NC = 2
NS = 16
L = 16
NW = NC * NS
EPW = E // NW
BLK = 80
NBLK = EPW // BLK
RPW = N // NS


def _pre_body(x_ref, w_ref, att_ref, h_ref, ab_ref):
    h = jnp.dot(x_ref[...], w_ref[...], preferred_element_type=jnp.float32)
    h_ref[...] = h
    ab_ref[...] = lax.dot_general(
        att_ref[...], h, (((1,), (1,)), ((), ())),
        preferred_element_type=jnp.float32)


def _precompute(x, W, att):
    return pl.pallas_call(
        _pre_body,
        out_shape=[
            jax.ShapeDtypeStruct((N, D), jnp.float32),
            jax.ShapeDtypeStruct((2, N), jnp.float32),
        ],
    )(x, W, att)


def _sc_edges(h, a_src, a_dst, src, dst):
    mesh = plsc.VectorSubcoreMesh(core_axis_name="c", subcore_axis_name="s")

    @functools.partial(
        pl.kernel,
        out_type=[
            jax.ShapeDtypeStruct((NC, N, D), jnp.float32),
            jax.ShapeDtypeStruct((NW, N), jnp.float32),
        ],
        mesh=mesh,
        scratch_types=[
            pltpu.VMEM((EPW,), jnp.int32),
            pltpu.VMEM((EPW,), jnp.int32),
            pltpu.VMEM((N,), jnp.float32),
            pltpu.VMEM((N,), jnp.float32),
            pltpu.VMEM((N,), jnp.float32),
            pltpu.VMEM((BLK, D), jnp.float32),
            pltpu.VMEM((BLK, D), jnp.float32),
            pltpu.VMEM((BLK,), jnp.float32),
            pltpu.VMEM((125, D), jnp.float32),
            pltpu.VMEM_SHARED((N, D), jnp.float32),
            pltpu.SemaphoreType.DMA,
            pltpu.SemaphoreType.DMA,
        ],
    )
    def body(h_hbm, asrc_hbm, adst_hbm, src_hbm, dst_hbm,
             acc_hbm, den_hbm,
             src_v, dst_v, asrc_v, adst_v, den_v,
             rows_a, rows_b, p_v, zbuf, acc_sh, sem_a, sem_b):
        cid = lax.axis_index("c")
        sid = lax.axis_index("s")
        wid = cid * NS + sid
        ebase = wid * EPW
        zero16 = jnp.zeros((L,), jnp.float32)

        @pl.loop(0, 125)
        def _(i):
            for j in range(D // L):
                zbuf[i, pl.ds(j * L, L)] = zero16

        for j in range(RPW // 125):
            pltpu.sync_copy(zbuf, acc_sh.at[pl.ds(sid * RPW + j * 125, 125)])

        @pl.loop(0, N, step=L)
        def _(i):
            den_v[pl.ds(i, L)] = zero16

        pltpu.sync_copy(asrc_hbm, asrc_v)
        pltpu.sync_copy(adst_hbm, adst_v)
        pltpu.sync_copy(src_hbm.at[pl.ds(ebase, EPW)], src_v)
        pltpu.sync_copy(dst_hbm.at[pl.ds(ebase, EPW)], dst_v)

        plsc.subcore_barrier()

        def issue(b, rows_ref, sem):
            pltpu.async_copy(
                h_hbm.at[src_v.at[pl.ds(b * BLK, BLK)]], rows_ref, sem)

        def process(b, rows_ref, sem):
            pltpu.make_async_copy(
                h_hbm.at[src_v.at[pl.ds(b * BLK, BLK)]], rows_ref, sem).wait()
            k = b * BLK
            for r in range(BLK // L):
                s16 = src_v[pl.ds(k + r * L, L)]
                d16 = dst_v[pl.ds(k + r * L, L)]
                a_s = plsc.load_gather(asrc_v, [s16])
                a_d = plsc.load_gather(adst_v, [d16])
                a_sd = plsc.load_gather(asrc_v, [d16])
                e = a_s + a_d
                e = jnp.maximum(e, 0.2 * e)
                c = a_sd + a_d
                c = jnp.maximum(c, 0.2 * c)
                p = jnp.exp(e - c)
                plsc.addupdate_scatter(den_v, [d16], p)
                p_v[pl.ds(r * L, L)] = p

            @pl.loop(0, BLK)
            def _(i):
                bp = plsc.load_gather(p_v, [jnp.full((L,), i, jnp.int32)])
                for j in range(D // L):
                    rows_ref[i, pl.ds(j * L, L)] = (
                        rows_ref[i, pl.ds(j * L, L)] * bp)

            for r in range(BLK // L):
                d16 = dst_v[pl.ds(k + r * L, L)]
                pltpu.sync_copy(rows_ref.at[pl.ds(r * L, L)],
                                acc_sh.at[d16], add=True)

        issue(0, rows_a, sem_a)

        @pl.loop(0, NBLK - 1, step=2)
        def _(i):
            issue(i + 1, rows_b, sem_b)
            process(i, rows_a, sem_a)
            issue(i + 2, rows_a, sem_a)
            process(i + 1, rows_b, sem_b)

        process(NBLK - 1, rows_a, sem_a)

        pltpu.sync_copy(den_v, den_hbm.at[wid])

        plsc.subcore_barrier()
        pltpu.sync_copy(acc_sh.at[pl.ds(sid * RPW, RPW)],
                        acc_hbm.at[cid].at[pl.ds(sid * RPW, RPW)])

    return body(h, a_src, a_dst, src, dst)


def _post_body(acc_ref, den_ref, h_ref, bias_ref, o_ref):
    den = jnp.sum(den_ref[...], axis=0) + jnp.float32(1.0)
    s = acc_ref[0] + acc_ref[1] + h_ref[...]
    o_ref[...] = jnp.maximum(s / den[:, None] + bias_ref[...], 0.0)


def _post(acc, den, h, bias2):
    return pl.pallas_call(
        _post_body,
        out_shape=jax.ShapeDtypeStruct((N, D), jnp.float32),
    )(acc, den, h, bias2)


def kernel(x, edge_index, edge_attr, W, att_src, att_dst, bias):
    src = edge_index[0].astype(jnp.int32)
    dst = edge_index[1].astype(jnp.int32)
    att = jnp.stack([att_src, att_dst])
    h, ab = _precompute(x, W, att)
    acc, den = _sc_edges(h, ab[0], ab[1], src, dst)
    return _post(acc, den, h, bias.reshape(1, D))


# speedup vs baseline: 30.0301x; 30.0301x over previous
import dataclasses
import functools

import jax
import jax.numpy as jnp
from jax import lax
from jax.experimental import pallas as pl
from jax.experimental.pallas import tpu as pltpu
from jax.experimental.pallas import tpu_sc as plsc

N = 10000
E = 320000
D = 128
DH = D // 2

NC = 2
NS = 16
L = 16
NW = NC * NS
EPS = E // NS
BLK = 80
NBLK = EPS // BLK


def _pre_body(x_ref, w_ref, att_ref, h_ref, ab_ref):
    h = jnp.dot(x_ref[...], w_ref[...], preferred_element_type=jnp.float32)
    h_ref[...] = h
    ab_ref[...] = lax.dot_general(
        att_ref[...], h, (((1,), (1,)), ((), ())),
        preferred_element_type=jnp.float32)


def _precompute(x, W, att):
    return pl.pallas_call(
        _pre_body,
        out_shape=[
            jax.ShapeDtypeStruct((N, D), jnp.float32),
            jax.ShapeDtypeStruct((2, N), jnp.float32),
        ],
    )(x, W, att)


def _sc_compiler_params():
    cp = pltpu.CompilerParams()
    fields = pltpu.CompilerParams.__dataclass_fields__
    if "needs_layout_passes" in fields:
        cp = dataclasses.replace(cp, needs_layout_passes=False)
    if "use_tc_tiling_on_sc" in fields:
        cp = dataclasses.replace(cp, use_tc_tiling_on_sc=False)
    return cp


def _sc_edges(h2, a_src, a_dst, src, dst):
    mesh = plsc.VectorSubcoreMesh(core_axis_name="c", subcore_axis_name="s")

    @functools.partial(
        pl.kernel,
        compiler_params=_sc_compiler_params(),
        out_type=[
            jax.ShapeDtypeStruct((NC, N, DH), jnp.float32),
            jax.ShapeDtypeStruct((NW, N), jnp.float32),
        ],
        mesh=mesh,
        scratch_types=[
            pltpu.VMEM((EPS,), jnp.int32),
            pltpu.VMEM((EPS,), jnp.int32),
            pltpu.VMEM((N,), jnp.float32),
            pltpu.VMEM((N,), jnp.float32),
            pltpu.VMEM((N,), jnp.float32),
            pltpu.VMEM((BLK, DH), jnp.float32),
            pltpu.VMEM((BLK, DH), jnp.float32),
            pltpu.VMEM((BLK,), jnp.float32),
            pltpu.VMEM((80, DH), jnp.float32),
            pltpu.VMEM_SHARED((N, DH), jnp.float32),
            pltpu.SemaphoreType.DMA,
            pltpu.SemaphoreType.DMA,
        ],
    )
    def body(h_hbm, asrc_hbm, adst_hbm, src_hbm, dst_hbm,
             acc_hbm, den_hbm,
             src_v, dst_v, asrc_v, adst_v, den_v,
             rows_a, rows_b, p_v, zbuf, acc_sh, sem_a, sem_b):
        cid = lax.axis_index("c")
        sid = lax.axis_index("s")
        wid = cid * NS + sid
        ebase = sid * EPS
        zero16 = jnp.zeros((L,), jnp.float32)

        @pl.loop(0, 80)
        def _(i):
            for j in range(DH // L):
                zbuf[i, pl.ds(j * L, L)] = zero16

        @pl.loop(0, 8)
        def _(g):
            t = g * NS + sid

            @pl.when(t < N // 80)
            def _():
                pltpu.sync_copy(zbuf, acc_sh.at[pl.ds(t * 80, 80)])

        @pl.loop(0, N, step=L)
        def _(i):
            den_v[pl.ds(i, L)] = zero16

        pltpu.sync_copy(asrc_hbm, asrc_v)
        pltpu.sync_copy(adst_hbm, adst_v)
        pltpu.sync_copy(src_hbm.at[pl.ds(ebase, EPS)], src_v)
        pltpu.sync_copy(dst_hbm.at[pl.ds(ebase, EPS)], dst_v)

        plsc.subcore_barrier()

        def issue(b, rows_ref, sem):
            pltpu.async_copy(
                h_hbm.at[cid].at[src_v.at[pl.ds(b * BLK, BLK)]],
                rows_ref, sem)

        def process(b, rows_ref, sem):
            pltpu.make_async_copy(
                h_hbm.at[cid].at[src_v.at[pl.ds(b * BLK, BLK)]],
                rows_ref, sem).wait()
            k = b * BLK
            for r in range(BLK // L):
                s16 = src_v[pl.ds(k + r * L, L)]
                d16 = dst_v[pl.ds(k + r * L, L)]
                a_s = plsc.load_gather(asrc_v, [s16])
                a_d = plsc.load_gather(adst_v, [d16])
                a_sd = plsc.load_gather(asrc_v, [d16])
                e = a_s + a_d
                e = jnp.maximum(e, 0.2 * e)
                c = a_sd + a_d
                c = jnp.maximum(c, 0.2 * c)
                p = jnp.exp(e - c)
                plsc.addupdate_scatter(den_v, [d16], p)
                p_v[pl.ds(r * L, L)] = p

            @pl.loop(0, BLK)
            def _(i):
                bp = plsc.load_gather(p_v, [jnp.full((L,), i, jnp.int32)])
                for j in range(DH // L):
                    rows_ref[i, pl.ds(j * L, L)] = (
                        rows_ref[i, pl.ds(j * L, L)] * bp)

            for r in range(BLK // L):
                d16 = dst_v[pl.ds(k + r * L, L)]
                pltpu.sync_copy(rows_ref.at[pl.ds(r * L, L)],
                                acc_sh.at[d16], add=True)

        issue(0, rows_a, sem_a)
        issue(1, rows_b, sem_b)

        @pl.loop(0, NBLK - 2, step=2)
        def _(i):
            process(i, rows_a, sem_a)
            issue(i + 2, rows_a, sem_a)
            process(i + 1, rows_b, sem_b)
            issue(i + 3, rows_b, sem_b)

        process(NBLK - 2, rows_a, sem_a)
        process(NBLK - 1, rows_b, sem_b)

        pltpu.sync_copy(den_v, den_hbm.at[wid])

        plsc.subcore_barrier()

        @pl.loop(0, 8)
        def _(g):
            t = g * NS + sid

            @pl.when(t < N // 80)
            def _():
                pltpu.sync_copy(acc_sh.at[pl.ds(t * 80, 80)],
                                acc_hbm.at[cid].at[pl.ds(t * 80, 80)])

    return body(h2, a_src, a_dst, src, dst)


def _post_body(acc_ref, den_ref, h_ref, bias_ref, o_ref):
    den = jnp.sum(den_ref[...], axis=0) * jnp.float32(0.5) + jnp.float32(1.0)
    s = jnp.concatenate([acc_ref[0], acc_ref[1]], axis=1) + h_ref[...]
    o_ref[...] = jnp.maximum(s / den[:, None] + bias_ref[...], 0.0)


def _post(acc, den, h, bias2):
    return pl.pallas_call(
        _post_body,
        out_shape=jax.ShapeDtypeStruct((N, D), jnp.float32),
    )(acc, den, h, bias2)


def kernel(x, edge_index, edge_attr, W, att_src, att_dst, bias):
    src = edge_index[0].astype(jnp.int32)
    dst = edge_index[1].astype(jnp.int32)
    att = jnp.stack([att_src, att_dst])
    h, ab = _precompute(x, W, att)
    h2 = jnp.stack([h[:, :DH], h[:, DH:]])
    acc, den = _sc_edges(h2, ab[0], ab[1], src, dst)
    return _post(acc, den, h, bias.reshape(1, D))


# speedup vs baseline: 38.0238x; 1.2662x over previous
import dataclasses
import functools

import jax
import jax.numpy as jnp
from jax import lax
from jax.experimental import pallas as pl
from jax.experimental.pallas import tpu as pltpu
from jax.experimental.pallas import tpu_sc as plsc

N = 10000
E = 320000
D = 128
DH = D // 2

NC = 2
NS = 16
L = 16
EPS = E // NS
BLK = 80
GSUB = 80
NBLK = EPS // BLK

RLO = 624
RHI = N - 15 * RLO


def _pre_body(x_ref, w_ref, att_ref, h_ref, ab_ref):
    h = jnp.dot(x_ref[...], w_ref[...], preferred_element_type=jnp.float32)
    h_ref[...] = h
    ab = lax.dot_general(
        att_ref[...], h, (((1,), (1,)), ((), ())),
        preferred_element_type=jnp.float32)
    s = ab[0] + ab[1]
    c = jnp.maximum(s, 0.2 * s)
    ab_ref[...] = jnp.concatenate(
        [ab, c[None], jnp.zeros_like(c)[None]], axis=0)


def _precompute(x, W, att):
    return pl.pallas_call(
        _pre_body,
        out_shape=[
            jax.ShapeDtypeStruct((N, D), jnp.float32),
            jax.ShapeDtypeStruct((4, N), jnp.float32),
        ],
    )(x, W, att)


def _sc_compiler_params():
    cp = pltpu.CompilerParams()
    fields = pltpu.CompilerParams.__dataclass_fields__
    if "needs_layout_passes" in fields:
        cp = dataclasses.replace(cp, needs_layout_passes=False)
    if "use_tc_tiling_on_sc" in fields:
        cp = dataclasses.replace(cp, use_tc_tiling_on_sc=False)
    return cp


def _sc_edges(h2, a_src, a_dst, a_c, src, dst):
    mesh = plsc.VectorSubcoreMesh(core_axis_name="c", subcore_axis_name="s")

    @functools.partial(
        pl.kernel,
        compiler_params=_sc_compiler_params(),
        out_type=[
            jax.ShapeDtypeStruct((NC, N, DH), jnp.float32),
            jax.ShapeDtypeStruct((NC * NS, N), jnp.float32),
        ],
        mesh=mesh,
        scratch_types=[
            pltpu.VMEM((EPS,), jnp.int32),
            pltpu.VMEM((EPS,), jnp.int32),
            pltpu.VMEM((N,), jnp.float32),
            pltpu.VMEM((N,), jnp.float32),
            pltpu.VMEM((N,), jnp.float32),
            pltpu.VMEM((BLK, DH), jnp.float32),
            pltpu.VMEM((BLK, DH), jnp.float32),
            pltpu.VMEM((BLK,), jnp.float32),
            pltpu.VMEM((80, DH), jnp.float32),
            pltpu.VMEM_SHARED((N, DH), jnp.float32),
            pltpu.SemaphoreType.DMA,
            pltpu.SemaphoreType.DMA,
            pltpu.SemaphoreType.DMA,
        ],
    )
    def body(h_hbm, asrc_hbm, adst_hbm, c_hbm, src_hbm, dst_hbm,
             acc_hbm, den_hbm,
             src_v, dst_v, asrc_v, adst_v, den_v,
             rows_a, rows_b, p_v, zbuf, acc_sh, sem_a, sem_b, sem_s):
        cid = lax.axis_index("c")
        sid = lax.axis_index("s")
        wid = cid * NS + sid
        ebase = sid * EPS
        zero16 = jnp.zeros((L,), jnp.float32)

        stage = [
            pltpu.async_copy(asrc_hbm, asrc_v, sem_s),
            pltpu.async_copy(adst_hbm, adst_v, sem_s),
            pltpu.async_copy(src_hbm.at[pl.ds(ebase, EPS)], src_v, sem_s),
            pltpu.async_copy(dst_hbm.at[pl.ds(ebase, EPS)], dst_v, sem_s),
        ]

        @pl.loop(0, 80)
        def _(i):
            for j in range(DH // L):
                zbuf[i, pl.ds(j * L, L)] = zero16

        @pl.loop(0, 8)
        def _(g):
            t = g * NS + sid

            @pl.when(t < N // 80)
            def _():
                pltpu.sync_copy(zbuf, acc_sh.at[pl.ds(t * 80, 80)])

        @pl.loop(0, N, step=L)
        def _(i):
            den_v[pl.ds(i, L)] = zero16

        for d in stage:
            d.wait()

        plsc.subcore_barrier()

        def issue(b, rows_ref, sem):
            k = b * BLK
            for g in range(BLK // GSUB):
                pltpu.async_copy(
                    h_hbm.at[cid].at[
                        src_v.at[pl.ds(k + g * GSUB, GSUB)]],
                    rows_ref.at[pl.ds(g * GSUB, GSUB)], sem)

        def process(b, rows_ref, sem):
            k = b * BLK
            for g in range(BLK // GSUB):
                pltpu.make_async_copy(
                    h_hbm.at[cid].at[
                        src_v.at[pl.ds(k + g * GSUB, GSUB)]],
                    rows_ref.at[pl.ds(g * GSUB, GSUB)], sem).wait()
            d16s = []
            for r in range(BLK // L):
                s16 = src_v[pl.ds(k + r * L, L)]
                d16 = dst_v[pl.ds(k + r * L, L)]
                a_s = plsc.load_gather(asrc_v, [s16])
                a_d = plsc.load_gather(adst_v, [d16])
                a_sd = plsc.load_gather(asrc_v, [d16])
                e = a_s + a_d
                e = jnp.maximum(e, 0.2 * e)
                c_d = a_sd + a_d
                c_d = jnp.maximum(c_d, 0.2 * c_d)
                p = jnp.exp(e - c_d)
                plsc.addupdate_scatter(den_v, [d16], p)
                p_v[pl.ds(r * L, L)] = p
                d16s.append(d16)

            @pl.loop(0, BLK, unroll=2)
            def _(i):
                bp = plsc.load_gather(p_v, [jnp.full((L,), i, jnp.int32)])
                for j in range(DH // L):
                    rows_ref[i, pl.ds(j * L, L)] = (
                        rows_ref[i, pl.ds(j * L, L)] * bp)

            descs = [
                pltpu.async_copy(rows_ref.at[pl.ds(r * L, L)],
                                 acc_sh.at[d16s[r]], sem_s, add=True)
                for r in range(BLK // L)
            ]
            for d in descs:
                d.wait()

        issue(0, rows_a, sem_a)
        issue(1, rows_b, sem_b)

        @pl.loop(0, NBLK - 2, step=2)
        def _(i):
            process(i, rows_a, sem_a)
            issue(i + 2, rows_a, sem_a)
            process(i + 1, rows_b, sem_b)
            issue(i + 3, rows_b, sem_b)

        process(NBLK - 2, rows_a, sem_a)
        process(NBLK - 1, rows_b, sem_b)

        pltpu.sync_copy(den_v, den_hbm.at[wid])

        plsc.subcore_barrier()

        @pl.when(sid < NS - 1)
        def _():
            base = sid * RLO
            pltpu.sync_copy(acc_sh.at[pl.ds(base, RLO)],
                            acc_hbm.at[cid].at[pl.ds(base, RLO)])

        @pl.when(sid == NS - 1)
        def _():
            base = (NS - 1) * RLO
            pltpu.sync_copy(acc_sh.at[pl.ds(base, RHI)],
                            acc_hbm.at[cid].at[pl.ds(base, RHI)])

    return body(h2, a_src, a_dst, a_c, src, dst)


def _post_body(acc_ref, den_ref, h_ref, bias_ref, o_ref):
    den = (jnp.sum(den_ref[...], axis=0) * jnp.float32(0.5)
           + jnp.float32(1.0))
    s = jnp.concatenate([acc_ref[0], acc_ref[1]], axis=1) + h_ref[...]
    o_ref[...] = jnp.maximum(s / den[:, None] + bias_ref[...], 0.0)


def _post(acc, den, h, bias2):
    return pl.pallas_call(
        _post_body,
        out_shape=jax.ShapeDtypeStruct((N, D), jnp.float32),
    )(acc, den, h, bias2)


def kernel(x, edge_index, edge_attr, W, att_src, att_dst, bias):
    src = edge_index[0].astype(jnp.int32)
    dst = edge_index[1].astype(jnp.int32)
    att = jnp.stack([att_src, att_dst])
    h, ab = _precompute(x, W, att)
    h2 = jnp.stack([h[:, :DH], h[:, DH:]])
    acc, den = _sc_edges(h2, ab[0], ab[1], ab[2], src, dst)
    return _post(acc, den, h, bias.reshape(1, D))


# speedup vs baseline: 38.9148x; 1.0234x over previous
import dataclasses
import functools

import jax
import jax.numpy as jnp
from jax import lax
from jax.experimental import pallas as pl
from jax.experimental.pallas import tpu as pltpu
from jax.experimental.pallas import tpu_sc as plsc

N = 10000
E = 320000
D = 128
DH = D // 2

NC = 2
NS = 16
L = 16
EPS = E // NS
BLK = 80
GSUB = 80
NBLK = EPS // BLK

RLO = 624
RHI = N - 15 * RLO


def _pre_body(x_ref, w_ref, att_ref, h_ref, h2_ref, ab_ref):
    h = jnp.dot(x_ref[...], w_ref[...], preferred_element_type=jnp.float32)
    h_ref[...] = h
    h2_ref[0] = h[:, :DH]
    h2_ref[1] = h[:, DH:]
    ab_ref[...] = lax.dot_general(
        att_ref[...], h, (((1,), (1,)), ((), ())),
        preferred_element_type=jnp.float32)


def _precompute(x, W, att):
    return pl.pallas_call(
        _pre_body,
        out_shape=[
            jax.ShapeDtypeStruct((N, D), jnp.float32),
            jax.ShapeDtypeStruct((NC, N, DH), jnp.float32),
            jax.ShapeDtypeStruct((2, N), jnp.float32),
        ],
    )(x, W, att)


def _sc_compiler_params():
    cp = pltpu.CompilerParams()
    fields = pltpu.CompilerParams.__dataclass_fields__
    if "needs_layout_passes" in fields:
        cp = dataclasses.replace(cp, needs_layout_passes=False)
    if "use_tc_tiling_on_sc" in fields:
        cp = dataclasses.replace(cp, use_tc_tiling_on_sc=False)
    return cp


def _sc_edges(h2, a_src, a_dst, src, dst):
    mesh = plsc.VectorSubcoreMesh(core_axis_name="c", subcore_axis_name="s")

    @functools.partial(
        pl.kernel,
        compiler_params=_sc_compiler_params(),
        out_type=[
            jax.ShapeDtypeStruct((NC, N, DH), jnp.float32),
            jax.ShapeDtypeStruct((NC * NS, N), jnp.float32),
        ],
        mesh=mesh,
        scratch_types=[
            pltpu.VMEM((EPS,), jnp.int32),
            pltpu.VMEM((EPS,), jnp.int32),
            pltpu.VMEM((N,), jnp.float32),
            pltpu.VMEM((N,), jnp.float32),
            pltpu.VMEM((N,), jnp.float32),
            pltpu.VMEM((BLK, DH), jnp.float32),
            pltpu.VMEM((BLK, DH), jnp.float32),
            pltpu.VMEM((BLK,), jnp.float32),
            pltpu.VMEM((80, DH), jnp.float32),
            pltpu.VMEM_SHARED((N, DH), jnp.float32),
            pltpu.SemaphoreType.DMA,
            pltpu.SemaphoreType.DMA,
            pltpu.SemaphoreType.DMA,
        ],
    )
    def body(h_hbm, asrc_hbm, adst_hbm, src_hbm, dst_hbm,
             acc_hbm, den_hbm,
             src_v, dst_v, asrc_v, adst_v, den_v,
             rows_a, rows_b, p_v, zbuf, acc_sh, sem_a, sem_b, sem_s):
        cid = lax.axis_index("c")
        sid = lax.axis_index("s")
        wid = cid * NS + sid
        ebase = sid * EPS
        zero16 = jnp.zeros((L,), jnp.float32)

        stage = [
            pltpu.async_copy(asrc_hbm, asrc_v, sem_s),
            pltpu.async_copy(adst_hbm, adst_v, sem_s),
            pltpu.async_copy(src_hbm.at[pl.ds(ebase, EPS)], src_v, sem_s),
            pltpu.async_copy(dst_hbm.at[pl.ds(ebase, EPS)], dst_v, sem_s),
        ]

        @pl.loop(0, 80)
        def _(i):
            for j in range(DH // L):
                zbuf[i, pl.ds(j * L, L)] = zero16

        @pl.loop(0, 8)
        def _(g):
            t = g * NS + sid

            @pl.when(t < N // 80)
            def _():
                pltpu.sync_copy(zbuf, acc_sh.at[pl.ds(t * 80, 80)])

        @pl.loop(0, N, step=L)
        def _(i):
            den_v[pl.ds(i, L)] = zero16

        for d in stage:
            d.wait()

        plsc.subcore_barrier()

        def issue(b, rows_ref, sem):
            k = b * BLK
            for g in range(BLK // GSUB):
                pltpu.async_copy(
                    h_hbm.at[cid].at[
                        src_v.at[pl.ds(k + g * GSUB, GSUB)]],
                    rows_ref.at[pl.ds(g * GSUB, GSUB)], sem)

        def process(b, rows_ref, sem):
            k = b * BLK
            for g in range(BLK // GSUB):
                pltpu.make_async_copy(
                    h_hbm.at[cid].at[
                        src_v.at[pl.ds(k + g * GSUB, GSUB)]],
                    rows_ref.at[pl.ds(g * GSUB, GSUB)], sem).wait()
            d16s = []
            for r in range(BLK // L):
                s16 = src_v[pl.ds(k + r * L, L)]
                d16 = dst_v[pl.ds(k + r * L, L)]
                a_s = plsc.load_gather(asrc_v, [s16])
                a_d = plsc.load_gather(adst_v, [d16])
                a_sd = plsc.load_gather(asrc_v, [d16])
                e = a_s + a_d
                e = jnp.maximum(e, 0.2 * e)
                c_d = a_sd + a_d
                c_d = jnp.maximum(c_d, 0.2 * c_d)
                p = jnp.exp(e - c_d)
                plsc.addupdate_scatter(den_v, [d16], p)
                p_v[pl.ds(r * L, L)] = p
                d16s.append(d16)

            @pl.loop(0, BLK, unroll=4)
            def _(i):
                bp = plsc.load_gather(p_v, [jnp.full((L,), i, jnp.int32)])
                for j in range(DH // L):
                    rows_ref[i, pl.ds(j * L, L)] = (
                        rows_ref[i, pl.ds(j * L, L)] * bp)

            descs = [
                pltpu.async_copy(rows_ref.at[pl.ds(r * L, L)],
                                 acc_sh.at[d16s[r]], sem_s, add=True)
                for r in range(BLK // L)
            ]
            for d in descs:
                d.wait()

        issue(0, rows_a, sem_a)
        issue(1, rows_b, sem_b)

        @pl.loop(0, NBLK - 2, step=2)
        def _(i):
            process(i, rows_a, sem_a)
            issue(i + 2, rows_a, sem_a)
            process(i + 1, rows_b, sem_b)
            issue(i + 3, rows_b, sem_b)

        process(NBLK - 2, rows_a, sem_a)
        process(NBLK - 1, rows_b, sem_b)

        pltpu.sync_copy(den_v, den_hbm.at[wid])

        plsc.subcore_barrier()

        @pl.when(sid < NS - 1)
        def _():
            base = sid * RLO
            pltpu.sync_copy(acc_sh.at[pl.ds(base, RLO)],
                            acc_hbm.at[cid].at[pl.ds(base, RLO)])

        @pl.when(sid == NS - 1)
        def _():
            base = (NS - 1) * RLO
            pltpu.sync_copy(acc_sh.at[pl.ds(base, RHI)],
                            acc_hbm.at[cid].at[pl.ds(base, RHI)])

    return body(h2, a_src, a_dst, src, dst)


def _post_body(acc_ref, den_ref, h_ref, bias_ref, o_ref):
    den = (jnp.sum(den_ref[...], axis=0) * jnp.float32(0.5)
           + jnp.float32(1.0))
    s = jnp.concatenate([acc_ref[0], acc_ref[1]], axis=1) + h_ref[...]
    o_ref[...] = jnp.maximum(s / den[:, None] + bias_ref[...], 0.0)


def _post(acc, den, h, bias2):
    return pl.pallas_call(
        _post_body,
        out_shape=jax.ShapeDtypeStruct((N, D), jnp.float32),
    )(acc, den, h, bias2)


def kernel(x, edge_index, edge_attr, W, att_src, att_dst, bias):
    src = edge_index[0].astype(jnp.int32)
    dst = edge_index[1].astype(jnp.int32)
    att = jnp.stack([att_src, att_dst])
    h, h2, ab = _precompute(x, W, att)
    acc, den = _sc_edges(h2, ab[0], ab[1], src, dst)
    return _post(acc, den, h, bias.reshape(1, D))
